# baseline (speedup 1.0000x reference)
import jax
import jax.numpy as jnp
from jax.experimental import pallas as pl

N = 10000


def _relu_body(x_ref, o_ref):
    o_ref[...] = jnp.maximum(x_ref[...], 0.0)


def kernel(x, edge_list, edge_attr, W, b, p):
    row = edge_list[0]
    col = edge_list[1]
    loop = jnp.arange(N, dtype=edge_list.dtype)
    row_f = jnp.concatenate([row, loop])
    col_f = jnp.concatenate([col, loop])
    w_f = jnp.concatenate([edge_attr, jnp.ones((N,), dtype=x.dtype)])
    deg = jax.ops.segment_sum(w_f, col_f, num_segments=N)
    deg_inv_sqrt = jnp.where(deg > 0, 1.0 / jnp.sqrt(deg), 0.0)
    norm = deg_inv_sqrt[row_f] * w_f * deg_inv_sqrt[col_f]
    xw = x @ W
    msg = xw[row_f] * norm[:, None]
    out = jax.ops.segment_sum(msg, col_f, num_segments=N) + b
    out = pl.pallas_call(
        _relu_body,
        out_shape=jax.ShapeDtypeStruct(out.shape, out.dtype),
    )(out)
    score = out @ p / jnp.linalg.norm(p)
    score = jnp.tanh(score)
    vals, perm = jax.lax.top_k(score, N)
    pooled = out[perm] * vals[:, None]
    return pooled


# speedup vs baseline: 10.4306x; 10.4306x over previous
import functools

import jax
import jax.numpy as jnp
from jax import lax
from jax.experimental import pallas as pl
from jax.experimental.pallas import tpu as pltpu
from jax.experimental.pallas import tpu_sc as plsc

N = 10000
NPAD = 10240
E = 320000
NC = 2
NS = 16
NW = NC * NS
EPT = E // NW
ROWS = 125
CW = 80
SLICE = NPAD // NS
PSL = NPAD // NW
PR = 4

_sc_mesh = plsc.VectorSubcoreMesh(core_axis_name="c", subcore_axis_name="s")
_sc_params = pltpu.CompilerParams(needs_layout_passes=False)


def _zero_fill(buf, nwords):
    def body(i, _):
        buf[pl.ds(i * 16, 16)] = jnp.zeros((16,), jnp.float32)
        return 0
    lax.fori_loop(0, nwords // 16, body, 0)


@functools.partial(
    pl.kernel,
    mesh=_sc_mesh,
    compiler_params=_sc_params,
    out_type=jax.ShapeDtypeStruct((NC, NPAD), jnp.float32),
    scratch_types=[
        pltpu.VMEM((ROWS, CW), jnp.int32),
        pltpu.VMEM((ROWS, CW), jnp.float32),
        pltpu.VMEM((SLICE,), jnp.float32),
        pltpu.VMEM_SHARED((NPAD,), jnp.float32),
    ],
)
def _deg_kernel(col_hbm, w_hbm, degp_hbm, col_v, w_v, z_v, deg_sh):
    c = lax.axis_index("c")
    s = lax.axis_index("s")
    wid = c * NS + s
    _zero_fill(z_v, SLICE)
    pltpu.sync_copy(z_v, deg_sh.at[pl.ds(s * SLICE, SLICE)])
    pltpu.sync_copy(col_hbm.at[wid], col_v)
    pltpu.sync_copy(w_hbm.at[wid], w_v)
    plsc.subcore_barrier()

    def srow(j, _):
        pltpu.sync_copy(w_v.at[j], deg_sh.at[col_v.at[j]], add=True)
        return 0
    lax.fori_loop(0, ROWS, srow, 0)
    plsc.subcore_barrier()
    pltpu.sync_copy(deg_sh.at[pl.ds(s * SLICE, SLICE)],
                    degp_hbm.at[c, pl.ds(s * SLICE, SLICE)])


def _mid_body(dpa_ref, dpb_ref, x_ref, w_ref, dis_ref, xw_ref):
    deg = dpa_ref[...] + dpb_ref[...] + 1.0
    dis_ref[...] = 1.0 / jnp.sqrt(deg)
    xw_ref[...] = jnp.dot(x_ref[...], w_ref[...],
                          preferred_element_type=jnp.float32)


def _mid_call(dpa, dpb, x_p, w_p):
    return pl.pallas_call(
        _mid_body,
        out_shape=(
            jax.ShapeDtypeStruct((NPAD // 128, 128), jnp.float32),
            jax.ShapeDtypeStruct((NPAD, 128), jnp.float32),
        ),
    )(dpa, dpb, x_p, w_p)


@functools.partial(
    pl.kernel,
    mesh=_sc_mesh,
    compiler_params=_sc_params,
    out_type=(
        jax.ShapeDtypeStruct((NC, NPAD), jnp.float32),
        jax.ShapeDtypeStruct((NC, NPAD), jnp.float32),
    ),
    scratch_types=[
        pltpu.VMEM((ROWS, CW), jnp.int32),
        pltpu.VMEM((ROWS, CW), jnp.int32),
        pltpu.VMEM((ROWS, CW), jnp.float32),
        pltpu.VMEM((ROWS, CW), jnp.float32),
        pltpu.VMEM((ROWS, CW), jnp.float32),
        pltpu.VMEM((NPAD,), jnp.float32),
        pltpu.VMEM((NPAD,), jnp.float32),
        pltpu.VMEM((NPAD,), jnp.float32),
        pltpu.VMEM((SLICE,), jnp.float32),
        pltpu.VMEM_SHARED((NPAD,), jnp.float32),
        pltpu.VMEM_SHARED((NPAD,), jnp.float32),
    ],
)
def _msg_kernel(row_hbm, col_hbm, w_hbm, dis_hbm, xw0_hbm, xw1_hbm,
                op0_hbm, op1_hbm,
                row_v, col_v, w_v, m0_v, m1_v, dis_v, x0_v, x1_v, z_v,
                o0_sh, o1_sh):
    c = lax.axis_index("c")
    s = lax.axis_index("s")
    wid = c * NS + s
    _zero_fill(z_v, SLICE)
    pltpu.sync_copy(z_v, o0_sh.at[pl.ds(s * SLICE, SLICE)])
    pltpu.sync_copy(z_v, o1_sh.at[pl.ds(s * SLICE, SLICE)])
    pltpu.sync_copy(row_hbm.at[wid], row_v)
    pltpu.sync_copy(col_hbm.at[wid], col_v)
    pltpu.sync_copy(w_hbm.at[wid], w_v)
    pltpu.sync_copy(dis_hbm, dis_v)
    pltpu.sync_copy(xw0_hbm, x0_v)
    pltpu.sync_copy(xw1_hbm, x1_v)
    plsc.subcore_barrier()

    def mrow(j, _):
        for k in range(CW // 16):
            sl = pl.ds(k * 16, 16)
            r = row_v[j, sl]
            cc = col_v[j, sl]
            wv = w_v[j, sl]
            dr = plsc.load_gather(dis_v, [r])
            dc = plsc.load_gather(dis_v, [cc])
            a0 = plsc.load_gather(x0_v, [r])
            a1 = plsc.load_gather(x1_v, [r])
            nrm = dr * wv * dc
            m0_v[j, sl] = a0 * nrm
            m1_v[j, sl] = a1 * nrm
        return 0
    lax.fori_loop(0, ROWS, mrow, 0)

    def srow(j, _):
        pltpu.sync_copy(m0_v.at[j], o0_sh.at[col_v.at[j]], add=True)
        pltpu.sync_copy(m1_v.at[j], o1_sh.at[col_v.at[j]], add=True)
        return 0
    lax.fori_loop(0, ROWS, srow, 0)
    plsc.subcore_barrier()
    pltpu.sync_copy(o0_sh.at[pl.ds(s * SLICE, SLICE)],
                    op0_hbm.at[c, pl.ds(s * SLICE, SLICE)])
    pltpu.sync_copy(o1_sh.at[pl.ds(s * SLICE, SLICE)],
                    op1_hbm.at[c, pl.ds(s * SLICE, SLICE)])


def _elem_body(pa0, pb0, pa1, pb1, xw0, xw1, dis, p_ref, b_ref,
               s_ref, q0_ref, q1_ref):
    d = dis[...]
    dis2 = d * d
    o0 = jnp.maximum(pa0[...] + pb0[...] + xw0[...] * dis2 + b_ref[0], 0.0)
    o1 = jnp.maximum(pa1[...] + pb1[...] + xw1[...] * dis2 + b_ref[1], 0.0)
    pnorm = jnp.sqrt(p_ref[0] * p_ref[0] + p_ref[1] * p_ref[1])
    score = (o0 * p_ref[0] + o1 * p_ref[1]) / pnorm
    sv = jnp.tanh(score)
    shape = s_ref.shape
    ii = (lax.broadcasted_iota(jnp.int32, shape, 0) * shape[1]
          + lax.broadcasted_iota(jnp.int32, shape, 1))
    valid = ii < N
    sv = jnp.where(valid, sv, -2.0)
    s_ref[...] = sv
    q0_ref[...] = jnp.where(valid, o0 * sv, 0.0)
    q1_ref[...] = jnp.where(valid, o1 * sv, 0.0)


def _elem_call(pa0, pb0, pa1, pb1, xw0, xw1, dis, p, b):
    shp = (NPAD // 128, 128)
    return pl.pallas_call(
        _elem_body,
        in_specs=[pl.BlockSpec(shp, lambda: (0, 0))] * 7
        + [pl.BlockSpec(memory_space=pltpu.SMEM)] * 2,
        out_specs=[pl.BlockSpec(shp, lambda: (0, 0))] * 3,
        out_shape=(jax.ShapeDtypeStruct(shp, jnp.float32),) * 3,
    )(pa0, pb0, pa1, pb1, xw0, xw1, dis, p, b)


_IB = 128
_JB = 1024


def _rank_body(srow_ref, scol_ref, rank_ref):
    i = pl.program_id(0)
    j = pl.program_id(1)
    a = srow_ref[...]
    c = scol_ref[...]
    diag = i // (_JB // _IB)

    @pl.when(j == 0)
    def _():
        rank_ref[...] = jnp.zeros((1, _IB), jnp.float32)

    @pl.when(j < diag)
    def _():
        cnt = jnp.sum(jnp.where(c >= a, 1.0, 0.0), axis=0, keepdims=True)
        rank_ref[...] += cnt

    @pl.when(j > diag)
    def _():
        cnt = jnp.sum(jnp.where(c > a, 1.0, 0.0), axis=0, keepdims=True)
        rank_ref[...] += cnt

    @pl.when(j == diag)
    def _():
        jj = j * _JB + lax.broadcasted_iota(jnp.int32, (_JB, _IB), 0)
        ii = i * _IB + lax.broadcasted_iota(jnp.int32, (_JB, _IB), 1)
        hit = jnp.logical_or(c > a, jnp.logical_and(c == a, jj < ii))
        rank_ref[...] += jnp.sum(jnp.where(hit, 1.0, 0.0),
                                 axis=0, keepdims=True)


def _rank_call(s_row, s_col):
    return pl.pallas_call(
        _rank_body,
        grid=(NPAD // _IB, NPAD // _JB),
        in_specs=[
            pl.BlockSpec((1, _IB), lambda i, j: (0, i)),
            pl.BlockSpec((_JB, 1), lambda i, j: (j, 0)),
        ],
        out_specs=pl.BlockSpec((1, _IB), lambda i, j: (0, i)),
        out_shape=jax.ShapeDtypeStruct((1, NPAD), jnp.float32),
    )(s_row, s_col)


@functools.partial(
    pl.kernel,
    mesh=_sc_mesh,
    compiler_params=_sc_params,
    out_type=(
        jax.ShapeDtypeStruct((NPAD,), jnp.float32),
        jax.ShapeDtypeStruct((NPAD,), jnp.float32),
    ),
    scratch_types=[
        pltpu.VMEM((PR, CW), jnp.int32),
        pltpu.VMEM((PR, CW), jnp.float32),
        pltpu.VMEM((PR, CW), jnp.float32),
    ],
)
def _scat_kernel(rank_hbm, q0_hbm, q1_hbm, p0_hbm, p1_hbm,
                 rank_v, q0_v, q1_v):
    c = lax.axis_index("c")
    s = lax.axis_index("s")
    wid = c * NS + s
    pltpu.sync_copy(rank_hbm.at[wid], rank_v)
    pltpu.sync_copy(q0_hbm.at[wid], q0_v)
    pltpu.sync_copy(q1_hbm.at[wid], q1_v)
    for j in range(PR):
        pltpu.sync_copy(q0_v.at[j], p0_hbm.at[rank_v.at[j]])
        pltpu.sync_copy(q1_v.at[j], p1_hbm.at[rank_v.at[j]])


def kernel(x, edge_list, edge_attr, W, b, p):
    row3 = edge_list[0].reshape(NW, ROWS, CW)
    col3 = edge_list[1].reshape(NW, ROWS, CW)
    w3 = edge_attr.reshape(NW, ROWS, CW)

    degp = _deg_kernel(col3, w3)

    x_p = jnp.pad(x, ((0, NPAD - N), (0, 0)))
    w_p = jnp.pad(W, ((0, 0), (0, 128 - W.shape[1])))
    shp = (NPAD // 128, 128)
    dis2d, xw = _mid_call(degp[0].reshape(shp), degp[1].reshape(shp),
                          x_p, w_p)
    dis = dis2d.reshape(NPAD)
    xw0 = xw[:, 0]
    xw1 = xw[:, 1]

    op0, op1 = _msg_kernel(row3, col3, w3, dis, xw0, xw1)

    s2d, q0, q1 = _elem_call(
        op0[0].reshape(shp), op0[1].reshape(shp),
        op1[0].reshape(shp), op1[1].reshape(shp),
        xw0.reshape(shp), xw1.reshape(shp), dis2d, p, b)

    s_flat = s2d.reshape(NPAD)
    rank_f = _rank_call(s_flat.reshape(1, NPAD), s_flat.reshape(NPAD, 1))
    rank = rank_f.reshape(NPAD).astype(jnp.int32)

    p0v, p1v = _scat_kernel(rank.reshape(NW, PR, CW),
                            q0.reshape(NPAD).reshape(NW, PR, CW),
                            q1.reshape(NPAD).reshape(NW, PR, CW))
    return jnp.stack([p0v[:N], p1v[:N]], axis=1)


# speedup vs baseline: 23.1839x; 2.2227x over previous
import functools

import jax
import jax.numpy as jnp
from jax import lax
from jax.experimental import pallas as pl
from jax.experimental.pallas import tpu as pltpu
from jax.experimental.pallas import tpu_sc as plsc

N = 10000
NPAD = 10240
E = 320000
NC = 2
NS = 16
NW = NC * NS
EPT = E // NW
ROWS = 125
CW = 80
SLICE = NPAD // NS
PSL = NPAD // NW
PR = 4

_sc_mesh = plsc.VectorSubcoreMesh(core_axis_name="c", subcore_axis_name="s")
_sc_params = pltpu.CompilerParams(needs_layout_passes=False)


def _zero_fill(buf, nwords):
    def body(i, _):
        buf[pl.ds(i * 16, 16)] = jnp.zeros((16,), jnp.float32)
        return 0
    lax.fori_loop(0, nwords // 16, body, 0)


@functools.partial(
    pl.kernel,
    mesh=_sc_mesh,
    compiler_params=_sc_params,
    out_type=jax.ShapeDtypeStruct((NC, NPAD), jnp.float32),
    scratch_types=[
        pltpu.VMEM((ROWS, CW), jnp.int32),
        pltpu.VMEM((ROWS, CW), jnp.float32),
        pltpu.VMEM((SLICE,), jnp.float32),
        pltpu.VMEM_SHARED((NPAD,), jnp.float32),
        pltpu.SemaphoreType.DMA,
    ],
)
def _deg_kernel(col_hbm, w_hbm, degp_hbm, col_v, w_v, z_v, deg_sh, sem):
    c = lax.axis_index("c")
    s = lax.axis_index("s")
    wid = c * NS + s
    d1 = pltpu.async_copy(col_hbm.at[wid], col_v, sem)
    d2 = pltpu.async_copy(w_hbm.at[wid], w_v, sem)
    _zero_fill(z_v, SLICE)
    pltpu.sync_copy(z_v, deg_sh.at[pl.ds(s * SLICE, SLICE)])
    d1.wait()
    d2.wait()
    plsc.subcore_barrier()

    def sgrp(g, _):
        ds = []
        for jj in range(5):
            j = g * 5 + jj
            ds.append(pltpu.async_copy(
                w_v.at[j], deg_sh.at[col_v.at[j]], sem, add=True))
        for d in ds:
            d.wait()
        return 0
    lax.fori_loop(0, ROWS // 5, sgrp, 0)
    plsc.subcore_barrier()
    pltpu.sync_copy(deg_sh.at[pl.ds(s * SLICE, SLICE)],
                    degp_hbm.at[c, pl.ds(s * SLICE, SLICE)])


def _mid_body(dpa_ref, dpb_ref, x_ref, w_ref, dis_ref, xw_ref):
    deg = dpa_ref[...] + dpb_ref[...] + 1.0
    dis_ref[...] = 1.0 / jnp.sqrt(deg)
    xw_ref[...] = jnp.dot(x_ref[...], w_ref[...],
                          preferred_element_type=jnp.float32)


def _mid_call(dpa, dpb, x_p, w_p):
    return pl.pallas_call(
        _mid_body,
        out_shape=(
            jax.ShapeDtypeStruct((NPAD // 128, 128), jnp.float32),
            jax.ShapeDtypeStruct((NPAD, 128), jnp.float32),
        ),
    )(dpa, dpb, x_p, w_p)


@functools.partial(
    pl.kernel,
    mesh=_sc_mesh,
    compiler_params=_sc_params,
    out_type=(
        jax.ShapeDtypeStruct((NC, NPAD), jnp.float32),
        jax.ShapeDtypeStruct((NC, NPAD), jnp.float32),
    ),
    scratch_types=[
        pltpu.VMEM((ROWS, CW), jnp.int32),
        pltpu.VMEM((ROWS, CW), jnp.int32),
        pltpu.VMEM((ROWS, CW), jnp.float32),
        pltpu.VMEM((ROWS, CW), jnp.float32),
        pltpu.VMEM((ROWS, CW), jnp.float32),
        pltpu.VMEM((NPAD,), jnp.float32),
        pltpu.VMEM((NPAD,), jnp.float32),
        pltpu.VMEM((NPAD,), jnp.float32),
        pltpu.VMEM((SLICE,), jnp.float32),
        pltpu.VMEM_SHARED((NPAD,), jnp.float32),
        pltpu.VMEM_SHARED((NPAD,), jnp.float32),
        pltpu.SemaphoreType.DMA,
    ],
)
def _msg_kernel(row_hbm, col_hbm, w_hbm, dis_hbm, xw0_hbm, xw1_hbm,
                op0_hbm, op1_hbm,
                row_v, col_v, w_v, m0_v, m1_v, dis_v, x0_v, x1_v, z_v,
                o0_sh, o1_sh, sem):
    c = lax.axis_index("c")
    s = lax.axis_index("s")
    wid = c * NS + s
    stage = [
        pltpu.async_copy(row_hbm.at[wid], row_v, sem),
        pltpu.async_copy(col_hbm.at[wid], col_v, sem),
        pltpu.async_copy(w_hbm.at[wid], w_v, sem),
        pltpu.async_copy(dis_hbm, dis_v, sem),
        pltpu.async_copy(xw0_hbm, x0_v, sem),
        pltpu.async_copy(xw1_hbm, x1_v, sem),
    ]
    _zero_fill(z_v, SLICE)
    pltpu.sync_copy(z_v, o0_sh.at[pl.ds(s * SLICE, SLICE)])
    pltpu.sync_copy(z_v, o1_sh.at[pl.ds(s * SLICE, SLICE)])
    for d in stage:
        d.wait()
    plsc.subcore_barrier()

    def mrow(j, _):
        for k in range(CW // 16):
            sl = pl.ds(k * 16, 16)
            r = row_v[j, sl]
            cc = col_v[j, sl]
            wv = w_v[j, sl]
            dr = plsc.load_gather(dis_v, [r])
            dc = plsc.load_gather(dis_v, [cc])
            a0 = plsc.load_gather(x0_v, [r])
            a1 = plsc.load_gather(x1_v, [r])
            nrm = dr * wv * dc
            m0_v[j, sl] = a0 * nrm
            m1_v[j, sl] = a1 * nrm
        return 0
    lax.fori_loop(0, ROWS, mrow, 0)

    def sgrp(g, _):
        ds = []
        for jj in range(5):
            j = g * 5 + jj
            ds.append(pltpu.async_copy(
                m0_v.at[j], o0_sh.at[col_v.at[j]], sem, add=True))
            ds.append(pltpu.async_copy(
                m1_v.at[j], o1_sh.at[col_v.at[j]], sem, add=True))
        for d in ds:
            d.wait()
        return 0
    lax.fori_loop(0, ROWS // 5, sgrp, 0)
    plsc.subcore_barrier()
    d1 = pltpu.async_copy(o0_sh.at[pl.ds(s * SLICE, SLICE)],
                          op0_hbm.at[c, pl.ds(s * SLICE, SLICE)], sem)
    d2 = pltpu.async_copy(o1_sh.at[pl.ds(s * SLICE, SLICE)],
                          op1_hbm.at[c, pl.ds(s * SLICE, SLICE)], sem)
    d1.wait()
    d2.wait()


def _elem_body(pa0, pb0, pa1, pb1, xw0, xw1, dis, p_ref, b_ref,
               s_ref, q0_ref, q1_ref):
    d = dis[...]
    dis2 = d * d
    o0 = jnp.maximum(pa0[...] + pb0[...] + xw0[...] * dis2 + b_ref[0], 0.0)
    o1 = jnp.maximum(pa1[...] + pb1[...] + xw1[...] * dis2 + b_ref[1], 0.0)
    pnorm = jnp.sqrt(p_ref[0] * p_ref[0] + p_ref[1] * p_ref[1])
    score = (o0 * p_ref[0] + o1 * p_ref[1]) / pnorm
    sv = jnp.tanh(score)
    shape = s_ref.shape
    ii = (lax.broadcasted_iota(jnp.int32, shape, 0) * shape[1]
          + lax.broadcasted_iota(jnp.int32, shape, 1))
    valid = ii < N
    sv = jnp.where(valid, sv, -2.0)
    s_ref[...] = sv
    q0_ref[...] = jnp.where(valid, o0 * sv, 0.0)
    q1_ref[...] = jnp.where(valid, o1 * sv, 0.0)


def _elem_call(pa0, pb0, pa1, pb1, xw0, xw1, dis, p, b):
    shp = (NPAD // 128, 128)
    return pl.pallas_call(
        _elem_body,
        in_specs=[pl.BlockSpec(shp, lambda: (0, 0))] * 7
        + [pl.BlockSpec(memory_space=pltpu.SMEM)] * 2,
        out_specs=[pl.BlockSpec(shp, lambda: (0, 0))] * 3,
        out_shape=(jax.ShapeDtypeStruct(shp, jnp.float32),) * 3,
    )(pa0, pb0, pa1, pb1, xw0, xw1, dis, p, b)


_IB = 256
_JB = 2048


def _rank_body(srow_ref, scol_ref, rank_ref):
    i = pl.program_id(0)
    j = pl.program_id(1)
    a = srow_ref[...]
    c = scol_ref[...]
    diag = i // (_JB // _IB)

    @pl.when(j == 0)
    def _():
        rank_ref[...] = jnp.zeros((1, _IB), jnp.float32)

    @pl.when(j < diag)
    def _():
        cnt = jnp.sum(jnp.where(c >= a, 1.0, 0.0), axis=0, keepdims=True)
        rank_ref[...] += cnt

    @pl.when(j > diag)
    def _():
        cnt = jnp.sum(jnp.where(c > a, 1.0, 0.0), axis=0, keepdims=True)
        rank_ref[...] += cnt

    @pl.when(j == diag)
    def _():
        jj = j * _JB + lax.broadcasted_iota(jnp.int32, (_JB, _IB), 0)
        ii = i * _IB + lax.broadcasted_iota(jnp.int32, (_JB, _IB), 1)
        hit = jnp.logical_or(c > a, jnp.logical_and(c == a, jj < ii))
        rank_ref[...] += jnp.sum(jnp.where(hit, 1.0, 0.0),
                                 axis=0, keepdims=True)


def _rank_call(s_row, s_col):
    return pl.pallas_call(
        _rank_body,
        grid=(NPAD // _IB, NPAD // _JB),
        in_specs=[
            pl.BlockSpec((1, _IB), lambda i, j: (0, i)),
            pl.BlockSpec((_JB, 1), lambda i, j: (j, 0)),
        ],
        out_specs=pl.BlockSpec((1, _IB), lambda i, j: (0, i)),
        out_shape=jax.ShapeDtypeStruct((1, NPAD), jnp.float32),
    )(s_row, s_col)


@functools.partial(
    pl.kernel,
    mesh=_sc_mesh,
    compiler_params=_sc_params,
    out_type=(
        jax.ShapeDtypeStruct((NPAD,), jnp.float32),
        jax.ShapeDtypeStruct((NPAD,), jnp.float32),
    ),
    scratch_types=[
        pltpu.VMEM((PR, CW), jnp.int32),
        pltpu.VMEM((PR, CW), jnp.float32),
        pltpu.VMEM((PR, CW), jnp.float32),
        pltpu.SemaphoreType.DMA,
    ],
)
def _scat_kernel(rank_hbm, q0_hbm, q1_hbm, p0_hbm, p1_hbm,
                 rank_v, q0_v, q1_v, sem):
    c = lax.axis_index("c")
    s = lax.axis_index("s")
    wid = c * NS + s
    stage = [
        pltpu.async_copy(rank_hbm.at[wid], rank_v, sem),
        pltpu.async_copy(q0_hbm.at[wid], q0_v, sem),
        pltpu.async_copy(q1_hbm.at[wid], q1_v, sem),
    ]
    for d in stage:
        d.wait()
    ds = []
    for j in range(PR):
        ds.append(pltpu.async_copy(q0_v.at[j], p0_hbm.at[rank_v.at[j]], sem))
        ds.append(pltpu.async_copy(q1_v.at[j], p1_hbm.at[rank_v.at[j]], sem))
    for d in ds:
        d.wait()


def kernel(x, edge_list, edge_attr, W, b, p):
    row3 = edge_list[0].reshape(NW, ROWS, CW)
    col3 = edge_list[1].reshape(NW, ROWS, CW)
    w3 = edge_attr.reshape(NW, ROWS, CW)

    degp = _deg_kernel(col3, w3)

    x_p = jnp.pad(x, ((0, NPAD - N), (0, 0)))
    w_p = jnp.pad(W, ((0, 0), (0, 128 - W.shape[1])))
    shp = (NPAD // 128, 128)
    dis2d, xw = _mid_call(degp[0].reshape(shp), degp[1].reshape(shp),
                          x_p, w_p)
    dis = dis2d.reshape(NPAD)
    xw0 = xw[:, 0]
    xw1 = xw[:, 1]

    op0, op1 = _msg_kernel(row3, col3, w3, dis, xw0, xw1)

    s2d, q0, q1 = _elem_call(
        op0[0].reshape(shp), op0[1].reshape(shp),
        op1[0].reshape(shp), op1[1].reshape(shp),
        xw0.reshape(shp), xw1.reshape(shp), dis2d, p, b)

    s_flat = s2d.reshape(NPAD)
    rank_f = _rank_call(s_flat.reshape(1, NPAD), s_flat.reshape(NPAD, 1))
    rank = rank_f.reshape(NPAD).astype(jnp.int32)

    p0v, p1v = _scat_kernel(rank.reshape(NW, PR, CW),
                            q0.reshape(NPAD).reshape(NW, PR, CW),
                            q1.reshape(NPAD).reshape(NW, PR, CW))
    return jnp.stack([p0v[:N], p1v[:N]], axis=1)
